# baseline (speedup 1.0000x reference)
import functools

import jax
import jax.numpy as jnp
from jax import lax
from jax.experimental import pallas as pl
from jax.experimental.pallas import tpu as pltpu
from jax.experimental.pallas import tpu_sc as plsc

_NUM_CLASSES = 100000
_FEAT_DIM = 64
_BATCH = 16384
_LAMBDA_C = 0.003

_NC = 2
_NS = 16
_NL = 16
_NW = _NC * _NS
_B_PER_W = _BATCH // _NW
_GROUPS = _FEAT_DIM // _NL
_CHUNK = 128
_NCHUNK = _B_PER_W // _CHUNK

_TBLK = 128
_OROWS = _TBLK // 2
_NBLK = _NUM_CLASSES // _TBLK
_REM = _NUM_CLASSES - _NBLK * _TBLK
_TROWS = (_NUM_CLASSES + _TBLK - 1) // _TBLK * _OROWS
_LAST_SLOT = _NBLK // _NW
_REM_WID = _NBLK - _LAST_SLOT * _NW


def _relayout(centers_t, tail16):
    mesh = plsc.VectorSubcoreMesh(core_axis_name="c", subcore_axis_name="s")

    @functools.partial(
        pl.kernel,
        mesh=mesh,
        out_type=jax.ShapeDtypeStruct((_TROWS, _TBLK), jnp.float32),
        compiler_params=pltpu.CompilerParams(needs_layout_passes=False),
        scratch_types=[
            pltpu.VMEM((_FEAT_DIM, _TBLK), jnp.float32),
            pltpu.VMEM((_FEAT_DIM, _TBLK), jnp.float32),
            pltpu.VMEM((_OROWS, _TBLK), jnp.float32),
            pltpu.VMEM((_OROWS, _TBLK), jnp.float32),
            pltpu.SemaphoreType.DMA,
            pltpu.SemaphoreType.DMA,
            pltpu.SemaphoreType.DMA,
            pltpu.SemaphoreType.DMA,
        ],
    )
    def k(tab_hbm, tail_hbm, out_hbm, in0, in1, ob0, ob1, sin0, sin1, sout0, sout1):
        wid = lax.axis_index("s") * _NC + lax.axis_index("c")
        ins = (in0, in1)
        obs = (ob0, ob1)
        sins = (sin0, sin1)
        souts = (sout0, sout1)
        lane = lax.iota(jnp.int32, _NL)
        perms = tuple((lane + i) & (_NL - 1) for i in range(_NL))
        r2s = tuple(p >> 1 for p in perms)
        c2s = tuple((p & 1) * _FEAT_DIM + lane for p in perms)

        def start_in(slot, b):
            bid = slot * _NW + wid
            pltpu.async_copy(
                tab_hbm.at[:, pl.ds(bid * _TBLK, _TBLK)], ins[b], sins[b])

        def wait_in(b):
            pltpu.make_async_copy(
                tab_hbm.at[:, pl.ds(0, _TBLK)], ins[b], sins[b]).wait()

        def start_out(slot, b):
            bid = slot * _NW + wid
            pltpu.async_copy(
                obs[b], out_hbm.at[pl.ds(bid * _OROWS, _OROWS), :], souts[b])

        def wait_out(b):
            pltpu.make_async_copy(
                obs[b], out_hbm.at[pl.ds(0, _OROWS), :], souts[b]).wait()

        def transpose(b):
            ib, ob = ins[b], obs[b]

            @plsc.parallel_loop(0, _GROUPS * (_TBLK // _NL), unroll=2)
            def _(x):
                dbase = (x >> 3) * _NL
                jbase = (x & 7) * _NL
                jhalf = (x & 7) * (_NL // 2)
                dlane = lane + dbase
                for i in range(_NL):
                    cols = perms[i] + jbase
                    v = plsc.load_gather(ib, [dlane, cols])
                    rows2 = r2s[i] + jhalf
                    cols2 = c2s[i] + dbase
                    plsc.store_scatter(ob, [rows2, cols2], v)

        start_in(0, 0)
        start_in(1, 1)

        @pl.loop(0, _LAST_SLOT, step=2)
        def _(s):
            for b in (0, 1):
                t = s + b
                wait_in(b)

                @pl.when((t + 2) * _NW + wid < _NBLK)
                def _():
                    start_in(t + 2, b)

                @pl.when(s > 0)
                def _():
                    wait_out(b)

                transpose(b)
                start_out(t, b)

        @pl.when(_LAST_SLOT * _NW + wid < _NBLK)
        def _():
            wait_in(0)
            wait_out(0)
            transpose(0)
            start_out(_LAST_SLOT, 0)

        wait_out(0)
        wait_out(1)

        @pl.when(wid == _REM_WID)
        def _():
            pltpu.sync_copy(tail_hbm, ob1.at[pl.ds(0, _REM // 2), :])
            pltpu.sync_copy(
                ob1.at[pl.ds(0, _REM // 2), :],
                out_hbm.at[pl.ds(_NBLK * _OROWS, _REM // 2), :])

    return k(centers_t, tail16)


def _partials(features, labels, table):
    mesh = plsc.VectorSubcoreMesh(core_axis_name="c", subcore_axis_name="s")

    @functools.partial(
        pl.kernel,
        mesh=mesh,
        out_type=jax.ShapeDtypeStruct((_NW, _NL), jnp.float32),
        compiler_params=pltpu.CompilerParams(needs_layout_passes=False),
        scratch_types=[
            pltpu.VMEM((_B_PER_W,), jnp.int32),
            pltpu.VMEM((_B_PER_W,), jnp.int32),
            pltpu.VMEM((_B_PER_W,), jnp.int32),
            pltpu.VMEM((_B_PER_W, _FEAT_DIM), jnp.float32),
            pltpu.VMEM((_CHUNK, 2 * _FEAT_DIM), jnp.float32),
            pltpu.VMEM((_CHUNK, 2 * _FEAT_DIM), jnp.float32),
            pltpu.VMEM((_NL,), jnp.float32),
            pltpu.SemaphoreType.DMA,
            pltpu.SemaphoreType.DMA,
            pltpu.SemaphoreType.DMA,
        ],
    )
    def k(feat_hbm, idx_hbm, table_hbm, out_hbm,
          idx_v, idx2_v, off_v, feat_v, rows0_v, rows1_v, acc_v,
          gsem0, gsem1, fsem):
        wid = lax.axis_index("s") * _NC + lax.axis_index("c")
        base = wid * _B_PER_W
        fcopy = pltpu.async_copy(
            feat_hbm.at[pl.ds(base, _B_PER_W), :], feat_v, fsem)
        pltpu.sync_copy(idx_hbm.at[pl.ds(base, _B_PER_W)], idx_v)

        @pl.loop(0, _B_PER_W, step=_NL)
        def _(r0):
            lab = idx_v[pl.ds(r0, _NL)]
            idx2_v[pl.ds(r0, _NL)] = lab >> 1
            off_v[pl.ds(r0, _NL)] = (lab & 1) * _FEAT_DIM

        rows_bufs = (rows0_v, rows1_v)
        gsems = (gsem0, gsem1)

        def gather(c):
            return pltpu.async_copy(
                table_hbm.at[idx2_v.at[pl.ds(c * _CHUNK, _CHUNK)]],
                rows_bufs[c % 2], gsems[c % 2])

        pending = gather(0)
        fcopy.wait()
        accs = tuple(jnp.zeros((_NL,), jnp.float32) for _ in range(_GROUPS))
        lane = lax.iota(jnp.int32, _NL)
        for c in range(_NCHUNK):
            pending.wait()
            if c + 1 < _NCHUNK:
                pending = gather(c + 1)
            rows_v = rows_bufs[c % 2]
            rbase = c * _CHUNK

            @plsc.parallel_loop(0, _CHUNK, unroll=2, carry=accs)
            def body(r, accs, rows_v=rows_v, rbase=rbase):
                rvec = jnp.full((_NL,), r, jnp.int32)
                colbase = plsc.load_gather(
                    off_v, [jnp.full((_NL,), rbase + r, jnp.int32)]) + lane
                out = []
                for g in range(_GROUPS):
                    f = feat_v[rbase + r, pl.ds(g * _NL, _NL)]
                    cc = plsc.load_gather(rows_v, [rvec, colbase + g * _NL])
                    d = f - cc
                    out.append(accs[g] + d * d)
                return tuple(out)

            accs = body
        acc_v[...] = accs[0] + accs[1] + accs[2] + accs[3]
        pltpu.sync_copy(acc_v, out_hbm.at[wid])

    return k(features, labels, table)


@jax.jit
def kernel(features, labels, centers):
    idx = labels.astype(jnp.int32)
    tail16 = centers[_NBLK * _TBLK:].reshape(_REM // 2, _TBLK)
    table = _relayout(centers.T, tail16)
    parts = _partials(features, idx, table)
    return (_LAMBDA_C / features.shape[0]) * jnp.sum(parts)
